# baseline (speedup 1.0000x reference)
import jax
import jax.numpy as jnp
from jax.experimental import pallas as pl
from jax.experimental.pallas import tpu as pltpu

L = 4
D = 768
H = 12
HD = 64
P = 256
G = 16
K = 8
PS = 14
TEMP = 0.1
MLP_D = 3072
SCALE = HD ** -0.5
NEG = -1e9
CIN = 3 * PS * PS


def _ln_rows(x, g, b):
    m = jnp.mean(x, axis=-1, keepdims=True)
    v = jnp.mean((x - m) ** 2, axis=-1, keepdims=True)
    return (x - m) * jax.lax.rsqrt(v + 1e-5) * g + b


def _l2n(x):
    n = jnp.sqrt(jnp.sum(x * x, axis=-1, keepdims=True))
    return x / jnp.maximum(n, 1e-12)


def _mm(a, b):
    return jnp.dot(a, b, preferred_element_type=jnp.float32)


def _mmT(a, b):
    return jax.lax.dot_general(a, b, (((1,), (1,)), ((), ())),
                               preferred_element_type=jnp.float32)


def _bf(x):
    return x.astype(jnp.bfloat16)


def _mmb(a, b):
    return jnp.dot(_bf(a), _bf(b), preferred_element_type=jnp.float32)


def _mmTb(a, b):
    return jax.lax.dot_general(_bf(a), _bf(b), (((1,), (1,)), ((), ())),
                               preferred_element_type=jnp.float32)


def _embed_kernel(xp_ref, w_ref, b_ref, pos_ref, out_ref):
    out_ref[0] = _mmb(xp_ref[0], w_ref[...]) + b_ref[...] + pos_ref[...]


def _layer_kernel(tokp_ref, tokc_ref, wq_ref, bq_ref, wk_ref, bk_ref, bias_ref,
                  qkvw_ref, qkvb_ref, projw_ref, projb_ref,
                  g1_ref, be1_ref, g2_ref, be2_ref,
                  w1_ref, mb1_ref, w2_ref, mb2_ref,
                  outp_ref, outc_ref):
    tokp = tokp_ref[0]
    tokc = tokc_ref[0]
    g1 = g1_ref[...]
    be1 = be1_ref[...]
    xn_p = _ln_rows(tokp, g1, be1)
    xn_c = _ln_rows(tokc, g1, be1)

    q = _l2n(_mm(xn_p, wq_ref[...]) + bq_ref[...])
    k = _l2n(_mm(xn_p, wk_ref[...]) + bk_ref[...])
    iota_q = jax.lax.broadcasted_iota(jnp.int32, (P, P), 1)
    iota_p = jax.lax.broadcasted_iota(jnp.int32, (P, P), 0)
    sc = _mmT(q, k) + bias_ref[...]
    work = jnp.where(iota_q == iota_p, NEG, sc)

    acc = jnp.zeros((P, P), jnp.float32)
    den = jnp.zeros((P, 1), jnp.float32)
    for _ in range(K):
        m = jnp.max(work, axis=-1, keepdims=True)
        oh = work == m
        work = jnp.where(oh, NEG, work)
        e = jnp.exp(m / TEMP)
        acc = acc + jnp.where(oh, e, 0.0)
        den = den + e
    rw_dense = acc / den
    routed = acc > 0.0

    qkvb = qkvb_ref[...]
    qkv_p = _mmb(xn_p, qkvw_ref[...]) + qkvb
    qkv_c = _mmb(xn_c, qkvw_ref[...]) + qkvb

    oc_parts = []
    op_parts = []
    for h in range(H):
        q0 = h * HD
        Qh = qkv_p[:, q0:q0 + HD]
        Kh = qkv_p[:, D + q0:D + q0 + HD]
        Vh = qkv_p[:, 2 * D + q0:2 * D + q0 + HD]
        qc = qkv_c[:, q0:q0 + HD]
        kc = qkv_c[:, D + q0:D + q0 + HD]
        vc = qkv_c[:, 2 * D + q0:2 * D + q0 + HD]

        lp = _mmT(qc, Kh) * SCALE
        ls = jnp.sum(qc * kc, axis=-1, keepdims=True) * SCALE
        mx = jnp.maximum(jnp.max(lp, axis=-1, keepdims=True), ls)
        ep = jnp.exp(lp - mx)
        ec = jnp.exp(ls - mx)
        denom_c = ec + jnp.sum(ep, axis=-1, keepdims=True)
        oc_parts.append((ec * vc + _mm(ep, Vh)) / denom_c)

        Zm = jnp.where(routed, _mmTb(Qh, Kh) * SCALE, NEG)
        zmax = jnp.max(Zm, axis=-1, keepdims=True)
        E = jnp.exp(Zm - zmax)
        s1 = jnp.sum(E, axis=-1, keepdims=True)
        Pw = E * rw_dense
        s2 = jnp.sum(Pw, axis=-1, keepdims=True)
        W = Pw / (s2 + 1e-6 * s1)
        op_parts.append(_mmb(W, Vh))

    oc = jnp.concatenate(oc_parts, axis=-1)
    op = jnp.concatenate(op_parts, axis=-1)

    projb = projb_ref[...]
    tokp1 = tokp + _mmb(op, projw_ref[...]) + projb
    tokc1 = tokc + _mmb(oc, projw_ref[...]) + projb

    g2 = g2_ref[...]
    be2 = be2_ref[...]
    mb1 = mb1_ref[...]
    mb2 = mb2_ref[...]
    xn2_p = _ln_rows(tokp1, g2, be2)
    xn2_c = _ln_rows(tokc1, g2, be2)
    h_p = jax.nn.gelu(_mmb(xn2_p, w1_ref[...]) + mb1)
    h_c = jax.nn.gelu(_mmb(xn2_c, w1_ref[...]) + mb1)
    outp_ref[0] = tokp1 + _mmb(h_p, w2_ref[...]) + mb2
    outc_ref[0] = tokc1 + _mmb(h_c, w2_ref[...]) + mb2


def _final_kernel(tokc_ref, g_ref, b_ref, out_ref):
    out_ref[...] = _ln_rows(tokc_ref[:, 0, :], g_ref[...], b_ref[...])


def _full(shape):
    nd = len(shape)
    return pl.BlockSpec(shape, lambda b: (0,) * nd)


def kernel(images, patch_w, patch_b, cls_token, pos_embed, router_wq, router_bq,
           router_wk, router_bk, pos_bias, qkv_w, qkv_b, proj_w, proj_b,
           ln1_g, ln1_b, ln2_g, ln2_b, mlp_w1, mlp_b1, mlp_w2, mlp_b2,
           lnf_g, lnf_b, interpret=False):
    B = images.shape[0]
    x = images.reshape(B, 3, G, PS, G, PS).transpose(0, 2, 4, 1, 3, 5)
    x = x.reshape(B, P, CIN)

    pos_p = pos_embed[0, 1:, :]
    tok_p = pl.pallas_call(
        _embed_kernel,
        grid=(B,),
        in_specs=[
            pl.BlockSpec((1, P, CIN), lambda b: (b, 0, 0)),
            _full((CIN, D)),
            _full((1, D)),
            _full((P, D)),
        ],
        out_specs=pl.BlockSpec((1, P, D), lambda b: (b, 0, 0)),
        out_shape=jax.ShapeDtypeStruct((B, P, D), jnp.float32),
        interpret=interpret,
    )(x, patch_w, patch_b.reshape(1, D), pos_p)

    tok_c = jnp.broadcast_to(cls_token[0] + pos_embed[0, :1, :], (B, 1, D))

    layer_call = pl.pallas_call(
        _layer_kernel,
        grid=(B,),
        in_specs=[
            pl.BlockSpec((1, P, D), lambda b: (b, 0, 0)),
            pl.BlockSpec((1, 1, D), lambda b: (b, 0, 0)),
            _full((D, D)), _full((1, D)),
            _full((D, D)), _full((1, D)),
            _full((P, P)),
            _full((D, 3 * D)), _full((1, 3 * D)),
            _full((D, D)), _full((1, D)),
            _full((1, D)), _full((1, D)),
            _full((1, D)), _full((1, D)),
            _full((D, MLP_D)), _full((1, MLP_D)),
            _full((MLP_D, D)), _full((1, D)),
        ],
        out_specs=[
            pl.BlockSpec((1, P, D), lambda b: (b, 0, 0)),
            pl.BlockSpec((1, 1, D), lambda b: (b, 0, 0)),
        ],
        out_shape=[
            jax.ShapeDtypeStruct((B, P, D), jnp.float32),
            jax.ShapeDtypeStruct((B, 1, D), jnp.float32),
        ],
        interpret=interpret,
    )

    for i in range(L):
        tok_p, tok_c = layer_call(
            tok_p, tok_c,
            router_wq[i], router_bq[i].reshape(1, D),
            router_wk[i], router_bk[i].reshape(1, D),
            pos_bias[i],
            qkv_w[i], qkv_b[i].reshape(1, 3 * D),
            proj_w[i], proj_b[i].reshape(1, D),
            ln1_g[i].reshape(1, D), ln1_b[i].reshape(1, D),
            ln2_g[i].reshape(1, D), ln2_b[i].reshape(1, D),
            mlp_w1[i], mlp_b1[i].reshape(1, MLP_D),
            mlp_w2[i], mlp_b2[i].reshape(1, D),
        )

    out = pl.pallas_call(
        _final_kernel,
        in_specs=[
            pl.BlockSpec((B, 1, D), lambda: (0, 0, 0)),
            pl.BlockSpec((1, D), lambda: (0, 0)),
            pl.BlockSpec((1, D), lambda: (0, 0)),
        ],
        out_specs=pl.BlockSpec((B, D), lambda: (0, 0)),
        out_shape=jax.ShapeDtypeStruct((B, D), jnp.float32),
        interpret=interpret,
    )(tok_c, lnf_g.reshape(1, D), lnf_b.reshape(1, D))
    return out


# speedup vs baseline: 1.1021x; 1.1021x over previous
import jax
import jax.numpy as jnp
from jax.experimental import pallas as pl
from jax.experimental.pallas import tpu as pltpu

L = 4
D = 768
H = 12
HD = 64
P = 256
G = 16
K = 8
PS = 14
TEMP = 0.1
MLP_D = 3072
SCALE = HD ** -0.5
NEG = -1e9
B = 4
CIN = 3 * PS * PS


def _ln_rows(x, g, b):
    m = jnp.mean(x, axis=-1, keepdims=True)
    v = jnp.mean((x - m) ** 2, axis=-1, keepdims=True)
    return (x - m) * jax.lax.rsqrt(v + 1e-5) * g + b


def _l2n(x):
    n = jnp.sqrt(jnp.sum(x * x, axis=-1, keepdims=True))
    return x / jnp.maximum(n, 1e-12)


def _mm(a, b):
    return jnp.dot(a, b, preferred_element_type=jnp.float32)


def _mmT(a, b):
    return jax.lax.dot_general(a, b, (((1,), (1,)), ((), ())),
                               preferred_element_type=jnp.float32)


def _bf(x):
    return x.astype(jnp.bfloat16)


def _mmb(a, b):
    return jnp.dot(_bf(a), b, preferred_element_type=jnp.float32)


def _mmTb(a, b):
    return jax.lax.dot_general(_bf(a), _bf(b), (((1,), (1,)), ((), ())),
                               preferred_element_type=jnp.float32)


def _embed_kernel(xp_ref, w_ref, b_ref, pos_ref, tokc0_ref, tokp_ref, tokc_ref):
    z = _mmb(xp_ref[...], w_ref[...]) + b_ref[...]
    pos = pos_ref[...]
    for b in range(B):
        tokp_ref[b] = z[b * P:(b + 1) * P, :] + pos
    tokc_ref[...] = tokc0_ref[...]


def _layer_kernel(tokp_ref, tokc_ref, wq_ref, bq_ref, wk_ref, bk_ref, bias_ref,
                  qkvw_ref, qkvb_ref, projw_ref, projb_ref,
                  g1_ref, be1_ref, g2_ref, be2_ref,
                  w1_ref, mb1_ref, w2_ref, mb2_ref,
                  outp_ref, outc_ref):
    tokp = tokp_ref[...].reshape(B * P, D)
    tokc = tokc_ref[...].reshape(B, D)
    g1 = g1_ref[...]
    be1 = be1_ref[...]
    xn_p = _ln_rows(tokp, g1, be1)
    xn_c = _ln_rows(tokc, g1, be1)

    q2 = _l2n(_mm(xn_p, wq_ref[...]) + bq_ref[...])
    k2 = _l2n(_mm(xn_p, wk_ref[...]) + bk_ref[...])

    qkvb = qkvb_ref[...]
    qkv_p = _mmb(xn_p, qkvw_ref[...]) + qkvb
    qkv_c = _mmb(xn_c, qkvw_ref[...]) + qkvb

    iota_q = jax.lax.broadcasted_iota(jnp.int32, (P, P), 1)
    iota_p = jax.lax.broadcasted_iota(jnp.int32, (P, P), 0)
    diag = iota_q == iota_p
    bias = bias_ref[...]

    op_rows = []
    oc_rows = []
    for b in range(B):
        r0 = b * P
        sc = _mmT(q2[r0:r0 + P, :], k2[r0:r0 + P, :]) + bias
        work = jnp.where(diag, NEG, sc)
        acc = jnp.zeros((P, P), jnp.float32)
        den = jnp.zeros((P, 1), jnp.float32)
        for _ in range(K):
            m = jnp.max(work, axis=-1, keepdims=True)
            oh = work == m
            work = jnp.where(oh, NEG, work)
            e = jnp.exp(m / TEMP)
            acc = acc + jnp.where(oh, e, 0.0)
            den = den + e
        rw_dense = acc / den
        routed = acc > 0.0

        oc_parts = []
        op_parts = []
        for h in range(H):
            q0 = h * HD
            Qh = qkv_p[r0:r0 + P, q0:q0 + HD]
            Kh = qkv_p[r0:r0 + P, D + q0:D + q0 + HD]
            Vh = qkv_p[r0:r0 + P, 2 * D + q0:2 * D + q0 + HD]
            qc = qkv_c[b:b + 1, q0:q0 + HD]
            kc = qkv_c[b:b + 1, D + q0:D + q0 + HD]
            vc = qkv_c[b:b + 1, 2 * D + q0:2 * D + q0 + HD]

            lp = _mmT(qc, Kh) * SCALE
            ls = jnp.sum(qc * kc, axis=-1, keepdims=True) * SCALE
            mx = jnp.maximum(jnp.max(lp, axis=-1, keepdims=True), ls)
            ep = jnp.exp(lp - mx)
            ec = jnp.exp(ls - mx)
            denom_c = ec + jnp.sum(ep, axis=-1, keepdims=True)
            oc_parts.append((ec * vc + _mm(ep, Vh)) / denom_c)

            Zm = jnp.where(routed, _mmTb(Qh, Kh) * SCALE, NEG)
            zmax = jnp.max(Zm, axis=-1, keepdims=True)
            E = jnp.exp(Zm - zmax)
            s1 = jnp.sum(E, axis=-1, keepdims=True)
            Pw = E * rw_dense
            s2 = jnp.sum(Pw, axis=-1, keepdims=True)
            W = Pw / (s2 + 1e-6 * s1)
            op_parts.append(_mmb(W, _bf(Vh)))

        op_rows.append(jnp.concatenate(op_parts, axis=-1))
        oc_rows.append(jnp.concatenate(oc_parts, axis=-1))

    op2 = jnp.concatenate(op_rows, axis=0)
    oc2 = jnp.concatenate(oc_rows, axis=0)

    projw = projw_ref[...]
    projb = projb_ref[...]
    tokp1 = tokp + _mmb(op2, projw) + projb
    tokc1 = tokc + _mmb(oc2, projw) + projb

    g2 = g2_ref[...]
    be2 = be2_ref[...]
    mb1 = mb1_ref[...]
    mb2 = mb2_ref[...]
    w1 = w1_ref[...]
    w2 = w2_ref[...]
    xn2_p = _ln_rows(tokp1, g2, be2)
    xn2_c = _ln_rows(tokc1, g2, be2)
    h_p = jax.nn.gelu(_mmb(xn2_p, w1) + mb1)
    h_c = jax.nn.gelu(_mmb(xn2_c, w1) + mb1)
    outp_ref[...] = (tokp1 + _mmb(h_p, w2) + mb2).reshape(B, P, D)
    outc_ref[...] = (tokc1 + _mmb(h_c, w2) + mb2).reshape(B, 1, D)


def _final_kernel(tokc_ref, g_ref, b_ref, out_ref):
    out_ref[...] = _ln_rows(tokc_ref[:, 0, :], g_ref[...], b_ref[...])


def kernel(images, patch_w, patch_b, cls_token, pos_embed, router_wq, router_bq,
           router_wk, router_bk, pos_bias, qkv_w, qkv_b, proj_w, proj_b,
           ln1_g, ln1_b, ln2_g, ln2_b, mlp_w1, mlp_b1, mlp_w2, mlp_b2,
           lnf_g, lnf_b, interpret=False):
    x = images.reshape(B, 3, G, PS, G, PS).transpose(0, 2, 4, 1, 3, 5)
    x2d = x.reshape(B * P, CIN)
    pos_p = pos_embed[0, 1:, :]
    tok_c0 = jnp.broadcast_to(cls_token[0] + pos_embed[0, :1, :], (B, 1, D))

    tok_p, tok_c = pl.pallas_call(
        _embed_kernel,
        out_shape=[
            jax.ShapeDtypeStruct((B, P, D), jnp.float32),
            jax.ShapeDtypeStruct((B, 1, D), jnp.float32),
        ],
        interpret=interpret,
    )(x2d, _bf(patch_w), patch_b.reshape(1, D), pos_p, tok_c0)

    layer_call = pl.pallas_call(
        _layer_kernel,
        out_shape=[
            jax.ShapeDtypeStruct((B, P, D), jnp.float32),
            jax.ShapeDtypeStruct((B, 1, D), jnp.float32),
        ],
        interpret=interpret,
    )

    for i in range(L):
        tok_p, tok_c = layer_call(
            tok_p, tok_c,
            router_wq[i], router_bq[i].reshape(1, D),
            router_wk[i], router_bk[i].reshape(1, D),
            pos_bias[i],
            _bf(qkv_w[i]), qkv_b[i].reshape(1, 3 * D),
            _bf(proj_w[i]), proj_b[i].reshape(1, D),
            ln1_g[i].reshape(1, D), ln1_b[i].reshape(1, D),
            ln2_g[i].reshape(1, D), ln2_b[i].reshape(1, D),
            _bf(mlp_w1[i]), mlp_b1[i].reshape(1, MLP_D),
            _bf(mlp_w2[i]), mlp_b2[i].reshape(1, D),
        )

    out = pl.pallas_call(
        _final_kernel,
        out_shape=jax.ShapeDtypeStruct((B, D), jnp.float32),
        interpret=interpret,
    )(tok_c, lnf_g.reshape(1, D), lnf_b.reshape(1, D))
    return out
